# baseline (speedup 1.0000x reference)
import jax
import jax.numpy as jnp
from jax import lax
from jax.experimental import pallas as pl
from jax.experimental.pallas import tpu as pltpu
from jax.experimental.pallas import tpu_sc as plsc

_N = 500000
_SUB = 30
_SG = 34
_CELLS = _SG * _SG * _SG

_NUM_CORES = 2
_NUM_SUBCORES = 16
_NW = _NUM_CORES * _NUM_SUBCORES
_LANES = 16

_CHUNK = 960
_NCHUNKS = 17
_PPT = _CHUNK * _NCHUNKS
_NPAD = _NW * _PPT
_GROUPS = _CHUNK // _LANES

_INV_SPACING = 30.5


def _weights(f):
    t = 1.0 - f
    s0 = t * t * t * (1.0 / 6.0)
    f2 = f * f
    f3 = f2 * f
    s1 = 0.5 * f3 - f2 + (2.0 / 3.0)
    s3 = f3 * (1.0 / 6.0)
    s2 = 1.0 - s0 - s1 - s3
    return s0, s1, s2, s3


def _sc_body(x_hbm, y_hbm, z_hbm, g0_hbm, g1_hbm, g2_hbm, o_hbm,
             g0_v, g1_v, g2_v,
             x0_v, y0_v, z0_v, x1_v, y1_v, z1_v,
             t0_v, t1_v,
             sgrid, sin0, sin1, sout0, sout1):
    wid = lax.axis_index("s") * _NUM_CORES + lax.axis_index("c")
    base = wid * _PPT

    ins = [(x0_v, y0_v, z0_v, sin0), (x1_v, y1_v, z1_v, sin1)]
    outs = [(t0_v, sout0), (t1_v, sout1)]

    def issue_in(k):
        xv, yv, zv, sem = ins[k % 2]
        off = base + k * _CHUNK
        return [
            pltpu.async_copy(x_hbm.at[pl.ds(off, _CHUNK)],
                             xv.at[pl.ds(0, _CHUNK)], sem),
            pltpu.async_copy(y_hbm.at[pl.ds(off, _CHUNK)],
                             yv.at[pl.ds(0, _CHUNK)], sem),
            pltpu.async_copy(z_hbm.at[pl.ds(off, _CHUNK)],
                             zv.at[pl.ds(0, _CHUNK)], sem),
        ]

    def issue_out(k):
        tv, sem = outs[k % 2]
        off3 = (base + k * _CHUNK) * 3
        return [pltpu.async_copy(tv, o_hbm.at[pl.ds(off3, _CHUNK * 3)], sem)]

    def compute_chunk(b):
        x_v, y_v, z_v, _ = ins[b]
        t_v, _ = outs[b]

        def group_prep(s):
            xv = x_v[pl.ds(s, _LANES)]
            yv = y_v[pl.ds(s, _LANES)]
            zv = z_v[pl.ds(s, _LANES)]

            u = xv * _INV_SPACING + _INV_SPACING
            v = yv * _INV_SPACING + _INV_SPACING
            w = zv * _INV_SPACING + _INV_SPACING
            ix = u.astype(jnp.int32)
            iy = v.astype(jnp.int32)
            iz = w.astype(jnp.int32)
            fu = u - ix.astype(jnp.float32)
            fv = v - iy.astype(jnp.float32)
            fw = w - iz.astype(jnp.float32)

            su = _weights(fu)
            sv = _weights(fv)
            sw = _weights(fw)

            cell = (ix * _SG + iy) * _SG + iz - (
                (_SUB * _SG + _SUB) * _SG + _SUB)
            return su + sv + sw + (cell,)

        def group_body(g, carry):
            s = g * _LANES
            prep, iota3 = carry[:13], carry[13]
            nxt = group_prep(s + _LANES)
            su, sv, sw = prep[0:4], prep[4:8], prep[8:12]
            cell = prep[12]

            a0 = [None] * 4
            a1 = [None] * 4
            a2 = [None] * 4
            for l in range(4):
                for m in range(4):
                    wlm = su[l] * sv[m]
                    row = cell + (l * _SG + m) * _SG
                    for n in range(4):
                        wt = wlm * sw[n]
                        idx = row + n
                        v0 = plsc.load_gather(g0_v, [idx])
                        v1 = plsc.load_gather(g1_v, [idx])
                        v2 = plsc.load_gather(g2_v, [idx])
                        if a0[n] is None:
                            a0[n] = wt * v0
                            a1[n] = wt * v1
                            a2[n] = wt * v2
                        else:
                            a0[n] = a0[n] + wt * v0
                            a1[n] = a1[n] + wt * v1
                            a2[n] = a2[n] + wt * v2

            i0 = iota3 + g * (3 * _LANES)
            plsc.store_scatter(t_v, [i0],
                               (a0[0] + a0[1]) + (a0[2] + a0[3]))
            plsc.store_scatter(t_v, [i0 + 1],
                               (a1[0] + a1[1]) + (a1[2] + a1[3]))
            plsc.store_scatter(t_v, [i0 + 2],
                               (a2[0] + a2[1]) + (a2[2] + a2[3]))
            return nxt + (iota3,)

        iota3 = lax.iota(jnp.int32, _LANES) * 3
        lax.fori_loop(0, _GROUPS, group_body, group_prep(0) + (iota3,))

    grid_copies = [
        pltpu.async_copy(g0_hbm, g0_v, sgrid),
        pltpu.async_copy(g1_hbm, g1_v, sgrid),
        pltpu.async_copy(g2_hbm, g2_v, sgrid),
    ]
    pending_in = issue_in(0)
    for h in grid_copies:
        h.wait()

    pending_out = {}
    for k in range(_NCHUNKS):
        for h in pending_in:
            h.wait()
        if k + 1 < _NCHUNKS:
            pending_in = issue_in(k + 1)
        if k - 2 in pending_out:
            for h in pending_out.pop(k - 2):
                h.wait()
        compute_chunk(k % 2)
        pending_out[k] = issue_out(k)
    for k in (_NCHUNKS - 2, _NCHUNKS - 1):
        for h in pending_out.pop(k):
            h.wait()


_sc_call = pl.kernel(
    _sc_body,
    out_type=jax.ShapeDtypeStruct((_NPAD * 3,), jnp.float32),
    mesh=plsc.VectorSubcoreMesh(
        core_axis_name="c", subcore_axis_name="s",
        num_cores=_NUM_CORES, num_subcores=_NUM_SUBCORES),
    scratch_types=[
        pltpu.VMEM((_CELLS,), jnp.float32),
        pltpu.VMEM((_CELLS,), jnp.float32),
        pltpu.VMEM((_CELLS,), jnp.float32),
        pltpu.VMEM((_CHUNK + _LANES,), jnp.float32),
        pltpu.VMEM((_CHUNK + _LANES,), jnp.float32),
        pltpu.VMEM((_CHUNK + _LANES,), jnp.float32),
        pltpu.VMEM((_CHUNK + _LANES,), jnp.float32),
        pltpu.VMEM((_CHUNK + _LANES,), jnp.float32),
        pltpu.VMEM((_CHUNK + _LANES,), jnp.float32),
        pltpu.VMEM((_CHUNK * 3,), jnp.float32),
        pltpu.VMEM((_CHUNK * 3,), jnp.float32),
        pltpu.SemaphoreType.DMA,
        pltpu.SemaphoreType.DMA,
        pltpu.SemaphoreType.DMA,
        pltpu.SemaphoreType.DMA,
        pltpu.SemaphoreType.DMA,
    ],
    compiler_params=pltpu.CompilerParams(needs_layout_passes=False),
)


def kernel(x, y, z, phi_x):
    sub = phi_x[_SUB:, _SUB:, _SUB:, :]
    g0 = sub[..., 0].reshape(_CELLS)
    g1 = sub[..., 1].reshape(_CELLS)
    g2 = sub[..., 2].reshape(_CELLS)
    pad = _NPAD - _N
    xp = jnp.concatenate([x, jnp.zeros((pad,), jnp.float32)])
    yp = jnp.concatenate([y, jnp.zeros((pad,), jnp.float32)])
    zp = jnp.concatenate([z, jnp.zeros((pad,), jnp.float32)])
    o = _sc_call(xp, yp, zp, g0, g1, g2)
    return o.reshape(_NPAD, 3)[:_N]


# speedup vs baseline: 2.1245x; 2.1245x over previous
import jax
import jax.numpy as jnp
from jax import lax
from jax.experimental import pallas as pl
from jax.experimental.pallas import tpu as pltpu
from jax.experimental.pallas import tpu_sc as plsc

_N = 500000
_SUB = 30
_SG = 34
_CELLS = _SG * _SG * _SG

_NUM_CORES = 2
_NUM_SUBCORES = 16
_NW = _NUM_CORES * _NUM_SUBCORES
_LANES = 16

_CHUNK = 960
_NCHUNKS = 17
_PPT = _CHUNK * _NCHUNKS
_NPAD = _NW * _PPT
_GROUPS = _CHUNK // _LANES

_INV_SPACING = 30.5


def _weights(f):
    t = 1.0 - f
    s0 = t * t * t * (1.0 / 6.0)
    f2 = f * f
    f3 = f2 * f
    s1 = 0.5 * f3 - f2 + (2.0 / 3.0)
    s3 = f3 * (1.0 / 6.0)
    s2 = 1.0 - s0 - s1 - s3
    return s0, s1, s2, s3


def _sc_body(x_hbm, y_hbm, z_hbm, g0_hbm, g1_hbm, g2_hbm,
             o0_hbm, o1_hbm, o2_hbm,
             g0_v, g1_v, g2_v,
             x0_v, y0_v, z0_v, x1_v, y1_v, z1_v,
             t00_v, t01_v, t02_v, t10_v, t11_v, t12_v,
             sgrid, sin0, sin1, sout0, sout1):
    wid = lax.axis_index("s") * _NUM_CORES + lax.axis_index("c")
    base = wid * _PPT

    ins = [(x0_v, y0_v, z0_v, sin0), (x1_v, y1_v, z1_v, sin1)]
    outs = [(t00_v, t01_v, t02_v, sout0), (t10_v, t11_v, t12_v, sout1)]

    def issue_in(k):
        xv, yv, zv, sem = ins[k % 2]
        off = base + k * _CHUNK
        return [
            pltpu.async_copy(x_hbm.at[pl.ds(off, _CHUNK)],
                             xv.at[pl.ds(0, _CHUNK)], sem),
            pltpu.async_copy(y_hbm.at[pl.ds(off, _CHUNK)],
                             yv.at[pl.ds(0, _CHUNK)], sem),
            pltpu.async_copy(z_hbm.at[pl.ds(off, _CHUNK)],
                             zv.at[pl.ds(0, _CHUNK)], sem),
        ]

    def issue_out(k):
        o0, o1, o2, sem = outs[k % 2]
        off = base + k * _CHUNK
        return [
            pltpu.async_copy(o0, o0_hbm.at[pl.ds(off, _CHUNK)], sem),
            pltpu.async_copy(o1, o1_hbm.at[pl.ds(off, _CHUNK)], sem),
            pltpu.async_copy(o2, o2_hbm.at[pl.ds(off, _CHUNK)], sem),
        ]

    def compute_chunk(b):
        x_v, y_v, z_v, _ = ins[b]
        t0_v, t1_v, t2_v, _ = outs[b]

        def group_prep(s):
            xv = x_v[pl.ds(s, _LANES)]
            yv = y_v[pl.ds(s, _LANES)]
            zv = z_v[pl.ds(s, _LANES)]

            u = xv * _INV_SPACING + _INV_SPACING
            v = yv * _INV_SPACING + _INV_SPACING
            w = zv * _INV_SPACING + _INV_SPACING
            ix = u.astype(jnp.int32)
            iy = v.astype(jnp.int32)
            iz = w.astype(jnp.int32)
            fu = u - ix.astype(jnp.float32)
            fv = v - iy.astype(jnp.float32)
            fw = w - iz.astype(jnp.float32)

            su = _weights(fu)
            sv = _weights(fv)
            sw = _weights(fw)

            cell = (ix * _SG + iy) * _SG + iz - (
                (_SUB * _SG + _SUB) * _SG + _SUB)
            return su + sv + sw + (cell,)

        def group_body(g, carry):
            s = g * _LANES
            prep = carry
            nxt = group_prep(s + _LANES)
            su, sv, sw = prep[0:4], prep[4:8], prep[8:12]
            cell = prep[12]

            a0 = [None] * 4
            a1 = [None] * 4
            a2 = [None] * 4
            for l in range(4):
                for m in range(4):
                    wlm = su[l] * sv[m]
                    row = cell + (l * _SG + m) * _SG
                    for n in range(4):
                        wt = wlm * sw[n]
                        idx = row + n
                        v0 = plsc.load_gather(g0_v, [idx])
                        v1 = plsc.load_gather(g1_v, [idx])
                        v2 = plsc.load_gather(g2_v, [idx])
                        if a0[n] is None:
                            a0[n] = wt * v0
                            a1[n] = wt * v1
                            a2[n] = wt * v2
                        else:
                            a0[n] = a0[n] + wt * v0
                            a1[n] = a1[n] + wt * v1
                            a2[n] = a2[n] + wt * v2

            t0_v[pl.ds(s, _LANES)] = (a0[0] + a0[1]) + (a0[2] + a0[3])
            t1_v[pl.ds(s, _LANES)] = (a1[0] + a1[1]) + (a1[2] + a1[3])
            t2_v[pl.ds(s, _LANES)] = (a2[0] + a2[1]) + (a2[2] + a2[3])
            return nxt

        lax.fori_loop(0, _GROUPS, group_body, group_prep(0))

    grid_copies = [
        pltpu.async_copy(g0_hbm, g0_v, sgrid),
        pltpu.async_copy(g1_hbm, g1_v, sgrid),
        pltpu.async_copy(g2_hbm, g2_v, sgrid),
    ]
    pending_in = issue_in(0)
    for h in grid_copies:
        h.wait()

    pending_out = {}
    for k in range(_NCHUNKS):
        for h in pending_in:
            h.wait()
        if k + 1 < _NCHUNKS:
            pending_in = issue_in(k + 1)
        if k - 2 in pending_out:
            for h in pending_out.pop(k - 2):
                h.wait()
        compute_chunk(k % 2)
        pending_out[k] = issue_out(k)
    for k in (_NCHUNKS - 2, _NCHUNKS - 1):
        for h in pending_out.pop(k):
            h.wait()


_sc_call = pl.kernel(
    _sc_body,
    out_type=(jax.ShapeDtypeStruct((_NPAD,), jnp.float32),
              jax.ShapeDtypeStruct((_NPAD,), jnp.float32),
              jax.ShapeDtypeStruct((_NPAD,), jnp.float32)),
    mesh=plsc.VectorSubcoreMesh(
        core_axis_name="c", subcore_axis_name="s",
        num_cores=_NUM_CORES, num_subcores=_NUM_SUBCORES),
    scratch_types=[
        pltpu.VMEM((_CELLS,), jnp.float32),
        pltpu.VMEM((_CELLS,), jnp.float32),
        pltpu.VMEM((_CELLS,), jnp.float32),
        pltpu.VMEM((_CHUNK + _LANES,), jnp.float32),
        pltpu.VMEM((_CHUNK + _LANES,), jnp.float32),
        pltpu.VMEM((_CHUNK + _LANES,), jnp.float32),
        pltpu.VMEM((_CHUNK + _LANES,), jnp.float32),
        pltpu.VMEM((_CHUNK + _LANES,), jnp.float32),
        pltpu.VMEM((_CHUNK + _LANES,), jnp.float32),
        pltpu.VMEM((_CHUNK,), jnp.float32),
        pltpu.VMEM((_CHUNK,), jnp.float32),
        pltpu.VMEM((_CHUNK,), jnp.float32),
        pltpu.VMEM((_CHUNK,), jnp.float32),
        pltpu.VMEM((_CHUNK,), jnp.float32),
        pltpu.VMEM((_CHUNK,), jnp.float32),
        pltpu.SemaphoreType.DMA,
        pltpu.SemaphoreType.DMA,
        pltpu.SemaphoreType.DMA,
        pltpu.SemaphoreType.DMA,
        pltpu.SemaphoreType.DMA,
    ],
    compiler_params=pltpu.CompilerParams(needs_layout_passes=False),
)


def kernel(x, y, z, phi_x):
    sub = phi_x[_SUB:, _SUB:, _SUB:, :]
    g0 = sub[..., 0].reshape(_CELLS)
    g1 = sub[..., 1].reshape(_CELLS)
    g2 = sub[..., 2].reshape(_CELLS)
    pad = _NPAD - _N
    xp = jnp.concatenate([x, jnp.zeros((pad,), jnp.float32)])
    yp = jnp.concatenate([y, jnp.zeros((pad,), jnp.float32)])
    zp = jnp.concatenate([z, jnp.zeros((pad,), jnp.float32)])
    o0, o1, o2 = _sc_call(xp, yp, zp, g0, g1, g2)
    return jnp.stack([o0[:_N], o1[:_N], o2[:_N]], axis=-1)


# speedup vs baseline: 2.1910x; 1.0313x over previous
import jax
import jax.numpy as jnp
from jax import lax
from jax.experimental import pallas as pl
from jax.experimental.pallas import tpu as pltpu
from jax.experimental.pallas import tpu_sc as plsc

_N = 500000
_SUB = 30
_SG = 34
_CELLS = _SG * _SG * _SG

_NUM_CORES = 2
_NUM_SUBCORES = 16
_NW = _NUM_CORES * _NUM_SUBCORES
_LANES = 16

_CHUNK = 960
_NCHUNKS = 17
_PPT = _CHUNK * _NCHUNKS
_NPAD = _NW * _PPT
_GROUPS = _CHUNK // _LANES

_INV_SPACING = 30.5


def _weights(f):
    t = 1.0 - f
    s0 = t * t * t * (1.0 / 6.0)
    f2 = f * f
    f3 = f2 * f
    s1 = 0.5 * f3 - f2 + (2.0 / 3.0)
    s3 = f3 * (1.0 / 6.0)
    s2 = 1.0 - s0 - s1 - s3
    return s0, s1, s2, s3


def _sc_body(x_hbm, y_hbm, z_hbm, g0_hbm, g1_hbm, g2_hbm,
             o0_hbm, o1_hbm, o2_hbm,
             g0_v, g1_v, g2_v,
             x0_v, y0_v, z0_v, x1_v, y1_v, z1_v,
             t00_v, t01_v, t02_v, t10_v, t11_v, t12_v,
             sgrid, sin0, sin1, sout0, sout1):
    wid = lax.axis_index("s") * _NUM_CORES + lax.axis_index("c")
    base = wid * _PPT

    ins = [(x0_v, y0_v, z0_v, sin0), (x1_v, y1_v, z1_v, sin1)]
    outs = [(t00_v, t01_v, t02_v, sout0), (t10_v, t11_v, t12_v, sout1)]

    def issue_in(k):
        xv, yv, zv, sem = ins[k % 2]
        off = jnp.minimum(base + k * _CHUNK, _N - _CHUNK)
        return [
            pltpu.async_copy(x_hbm.at[pl.ds(off, _CHUNK)],
                             xv.at[pl.ds(0, _CHUNK)], sem),
            pltpu.async_copy(y_hbm.at[pl.ds(off, _CHUNK)],
                             yv.at[pl.ds(0, _CHUNK)], sem),
            pltpu.async_copy(z_hbm.at[pl.ds(off, _CHUNK)],
                             zv.at[pl.ds(0, _CHUNK)], sem),
        ]

    def issue_out(k):
        o0, o1, o2, sem = outs[k % 2]
        off = jnp.minimum(base + k * _CHUNK, _N - _CHUNK)
        return [
            pltpu.async_copy(o0, o0_hbm.at[pl.ds(off, _CHUNK)], sem),
            pltpu.async_copy(o1, o1_hbm.at[pl.ds(off, _CHUNK)], sem),
            pltpu.async_copy(o2, o2_hbm.at[pl.ds(off, _CHUNK)], sem),
        ]

    def compute_chunk(b):
        x_v, y_v, z_v, _ = ins[b]
        t0_v, t1_v, t2_v, _ = outs[b]

        def group_prep(s):
            xv = x_v[pl.ds(s, _LANES)]
            yv = y_v[pl.ds(s, _LANES)]
            zv = z_v[pl.ds(s, _LANES)]

            u = xv * _INV_SPACING + _INV_SPACING
            v = yv * _INV_SPACING + _INV_SPACING
            w = zv * _INV_SPACING + _INV_SPACING
            ix = u.astype(jnp.int32)
            iy = v.astype(jnp.int32)
            iz = w.astype(jnp.int32)
            fu = u - ix.astype(jnp.float32)
            fv = v - iy.astype(jnp.float32)
            fw = w - iz.astype(jnp.float32)

            su = _weights(fu)
            sv = _weights(fv)
            sw = _weights(fw)

            cell = (ix * _SG + iy) * _SG + iz - (
                (_SUB * _SG + _SUB) * _SG + _SUB)
            return su + sv + sw + (cell,)

        def group_body(g, carry):
            s = g * _LANES
            prep = carry
            nxt = group_prep(s + _LANES)
            su, sv, sw = prep[0:4], prep[4:8], prep[8:12]
            cell = prep[12]

            a0 = [None] * 4
            a1 = [None] * 4
            a2 = [None] * 4
            for l in range(4):
                for m in range(4):
                    wlm = su[l] * sv[m]
                    row = cell + (l * _SG + m) * _SG
                    for n in range(4):
                        wt = wlm * sw[n]
                        idx = row + n
                        v0 = plsc.load_gather(g0_v, [idx])
                        v1 = plsc.load_gather(g1_v, [idx])
                        v2 = plsc.load_gather(g2_v, [idx])
                        if a0[n] is None:
                            a0[n] = wt * v0
                            a1[n] = wt * v1
                            a2[n] = wt * v2
                        else:
                            a0[n] = a0[n] + wt * v0
                            a1[n] = a1[n] + wt * v1
                            a2[n] = a2[n] + wt * v2

            t0_v[pl.ds(s, _LANES)] = (a0[0] + a0[1]) + (a0[2] + a0[3])
            t1_v[pl.ds(s, _LANES)] = (a1[0] + a1[1]) + (a1[2] + a1[3])
            t2_v[pl.ds(s, _LANES)] = (a2[0] + a2[1]) + (a2[2] + a2[3])
            return nxt

        lax.fori_loop(0, _GROUPS, group_body, group_prep(0))

    grid_copies = [
        pltpu.async_copy(g0_hbm, g0_v, sgrid),
        pltpu.async_copy(g1_hbm, g1_v, sgrid),
        pltpu.async_copy(g2_hbm, g2_v, sgrid),
    ]
    pending_in = issue_in(0)
    for h in grid_copies:
        h.wait()

    pending_out = {}
    for k in range(_NCHUNKS):
        for h in pending_in:
            h.wait()
        if k + 1 < _NCHUNKS:
            pending_in = issue_in(k + 1)
        if k - 2 in pending_out:
            for h in pending_out.pop(k - 2):
                h.wait()
        compute_chunk(k % 2)
        pending_out[k] = issue_out(k)
    for k in sorted(pending_out):
        for h in pending_out.pop(k):
            h.wait()


_sc_call = pl.kernel(
    _sc_body,
    out_type=(jax.ShapeDtypeStruct((_N,), jnp.float32),
              jax.ShapeDtypeStruct((_N,), jnp.float32),
              jax.ShapeDtypeStruct((_N,), jnp.float32)),
    mesh=plsc.VectorSubcoreMesh(
        core_axis_name="c", subcore_axis_name="s",
        num_cores=_NUM_CORES, num_subcores=_NUM_SUBCORES),
    scratch_types=[
        pltpu.VMEM((_CELLS,), jnp.float32),
        pltpu.VMEM((_CELLS,), jnp.float32),
        pltpu.VMEM((_CELLS,), jnp.float32),
        pltpu.VMEM((_CHUNK + _LANES,), jnp.float32),
        pltpu.VMEM((_CHUNK + _LANES,), jnp.float32),
        pltpu.VMEM((_CHUNK + _LANES,), jnp.float32),
        pltpu.VMEM((_CHUNK + _LANES,), jnp.float32),
        pltpu.VMEM((_CHUNK + _LANES,), jnp.float32),
        pltpu.VMEM((_CHUNK + _LANES,), jnp.float32),
        pltpu.VMEM((_CHUNK,), jnp.float32),
        pltpu.VMEM((_CHUNK,), jnp.float32),
        pltpu.VMEM((_CHUNK,), jnp.float32),
        pltpu.VMEM((_CHUNK,), jnp.float32),
        pltpu.VMEM((_CHUNK,), jnp.float32),
        pltpu.VMEM((_CHUNK,), jnp.float32),
        pltpu.SemaphoreType.DMA,
        pltpu.SemaphoreType.DMA,
        pltpu.SemaphoreType.DMA,
        pltpu.SemaphoreType.DMA,
        pltpu.SemaphoreType.DMA,
    ],
    compiler_params=pltpu.CompilerParams(needs_layout_passes=False),
)


def kernel(x, y, z, phi_x):
    sub = phi_x[_SUB:, _SUB:, _SUB:, :]
    g0 = sub[..., 0].reshape(_CELLS)
    g1 = sub[..., 1].reshape(_CELLS)
    g2 = sub[..., 2].reshape(_CELLS)
    o0, o1, o2 = _sc_call(x, y, z, g0, g1, g2)
    return jnp.stack([o0, o1, o2], axis=-1)


# speedup vs baseline: 2.2513x; 1.0275x over previous
import jax
import jax.numpy as jnp
from jax import lax
from jax.experimental import pallas as pl
from jax.experimental.pallas import tpu as pltpu
from jax.experimental.pallas import tpu_sc as plsc

_N = 500000
_SUB = 30
_SG = 34
_CELLS = _SG * _SG * _SG

_NUM_CORES = 2
_NUM_SUBCORES = 16
_NW = _NUM_CORES * _NUM_SUBCORES
_LANES = 16

_CHUNK = 960
_NCHUNKS = 17
_PPT = _CHUNK * _NCHUNKS
_NPAD = _NW * _PPT
_GROUPS = _CHUNK // _LANES

_INV_SPACING = 30.5


def _weights(f):
    t = 1.0 - f
    s0 = t * t * t * (1.0 / 6.0)
    f2 = f * f
    f3 = f2 * f
    s1 = 0.5 * f3 - f2 + (2.0 / 3.0)
    s3 = f3 * (1.0 / 6.0)
    s2 = 1.0 - s0 - s1 - s3
    return s0, s1, s2, s3


def _sc_body(x_hbm, y_hbm, z_hbm, g0_hbm, g1_hbm, g2_hbm,
             o0_hbm, o1_hbm, o2_hbm,
             g0_v, g1_v, g2_v,
             x0_v, y0_v, z0_v, x1_v, y1_v, z1_v,
             t00_v, t01_v, t02_v, t10_v, t11_v, t12_v,
             sgrid, sin0, sin1, sout0, sout1):
    wid = lax.axis_index("s") * _NUM_CORES + lax.axis_index("c")
    base = wid * _PPT

    ins = [(x0_v, y0_v, z0_v, sin0), (x1_v, y1_v, z1_v, sin1)]
    outs = [(t00_v, t01_v, t02_v, sout0), (t10_v, t11_v, t12_v, sout1)]

    def issue_in(k):
        xv, yv, zv, sem = ins[k % 2]
        off = jnp.minimum(base + k * _CHUNK, _N - _CHUNK)
        return [
            pltpu.async_copy(x_hbm.at[pl.ds(off, _CHUNK)],
                             xv.at[pl.ds(0, _CHUNK)], sem),
            pltpu.async_copy(y_hbm.at[pl.ds(off, _CHUNK)],
                             yv.at[pl.ds(0, _CHUNK)], sem),
            pltpu.async_copy(z_hbm.at[pl.ds(off, _CHUNK)],
                             zv.at[pl.ds(0, _CHUNK)], sem),
        ]

    def issue_out(k):
        o0, o1, o2, sem = outs[k % 2]
        off = jnp.minimum(base + k * _CHUNK, _N - _CHUNK)
        return [
            pltpu.async_copy(o0, o0_hbm.at[pl.ds(off, _CHUNK)], sem),
            pltpu.async_copy(o1, o1_hbm.at[pl.ds(off, _CHUNK)], sem),
            pltpu.async_copy(o2, o2_hbm.at[pl.ds(off, _CHUNK)], sem),
        ]

    def compute_chunk(b):
        x_v, y_v, z_v, _ = ins[b]
        t0_v, t1_v, t2_v, _ = outs[b]

        def group_prep(s):
            xv = x_v[pl.ds(s, _LANES)]
            yv = y_v[pl.ds(s, _LANES)]
            zv = z_v[pl.ds(s, _LANES)]

            u = xv * _INV_SPACING + _INV_SPACING
            v = yv * _INV_SPACING + _INV_SPACING
            w = zv * _INV_SPACING + _INV_SPACING
            ix = u.astype(jnp.int32)
            iy = v.astype(jnp.int32)
            iz = w.astype(jnp.int32)
            fu = u - ix.astype(jnp.float32)
            fv = v - iy.astype(jnp.float32)
            fw = w - iz.astype(jnp.float32)

            su = _weights(fu)
            sv = _weights(fv)
            sw = _weights(fw)

            cell = (ix * _SG + iy) * _SG + iz - (
                (_SUB * _SG + _SUB) * _SG + _SUB)
            return su + sv + sw + (cell,)

        def group_body(g, carry):
            s = g * _LANES
            prep = carry
            nxt = group_prep(s + _LANES)
            su, sv, sw = prep[0:4], prep[4:8], prep[8:12]
            cell = prep[12]

            a0 = a1 = a2 = None
            for l in range(4):
                for m in range(4):
                    wlm = su[l] * sv[m]
                    row = cell + (l * _SG + m) * _SG
                    for n in range(4):
                        wt = wlm * sw[n]
                        idx = row + n
                        v0 = plsc.load_gather(g0_v, [idx])
                        v1 = plsc.load_gather(g1_v, [idx])
                        v2 = plsc.load_gather(g2_v, [idx])
                        if a0 is None:
                            a0 = wt * v0
                            a1 = wt * v1
                            a2 = wt * v2
                        else:
                            a0 = a0 + wt * v0
                            a1 = a1 + wt * v1
                            a2 = a2 + wt * v2

            t0_v[pl.ds(s, _LANES)] = a0
            t1_v[pl.ds(s, _LANES)] = a1
            t2_v[pl.ds(s, _LANES)] = a2
            return nxt

        lax.fori_loop(0, _GROUPS, group_body, group_prep(0))

    grid_copies = [
        pltpu.async_copy(g0_hbm, g0_v, sgrid),
        pltpu.async_copy(g1_hbm, g1_v, sgrid),
        pltpu.async_copy(g2_hbm, g2_v, sgrid),
    ]
    pending_in = issue_in(0)
    for h in grid_copies:
        h.wait()

    pending_out = {}
    for k in range(_NCHUNKS):
        for h in pending_in:
            h.wait()
        if k + 1 < _NCHUNKS:
            pending_in = issue_in(k + 1)
        if k - 2 in pending_out:
            for h in pending_out.pop(k - 2):
                h.wait()
        compute_chunk(k % 2)
        pending_out[k] = issue_out(k)
    for k in sorted(pending_out):
        for h in pending_out.pop(k):
            h.wait()


_sc_call = pl.kernel(
    _sc_body,
    out_type=(jax.ShapeDtypeStruct((_N,), jnp.float32),
              jax.ShapeDtypeStruct((_N,), jnp.float32),
              jax.ShapeDtypeStruct((_N,), jnp.float32)),
    mesh=plsc.VectorSubcoreMesh(
        core_axis_name="c", subcore_axis_name="s",
        num_cores=_NUM_CORES, num_subcores=_NUM_SUBCORES),
    scratch_types=[
        pltpu.VMEM((_CELLS,), jnp.float32),
        pltpu.VMEM((_CELLS,), jnp.float32),
        pltpu.VMEM((_CELLS,), jnp.float32),
        pltpu.VMEM((_CHUNK + _LANES,), jnp.float32),
        pltpu.VMEM((_CHUNK + _LANES,), jnp.float32),
        pltpu.VMEM((_CHUNK + _LANES,), jnp.float32),
        pltpu.VMEM((_CHUNK + _LANES,), jnp.float32),
        pltpu.VMEM((_CHUNK + _LANES,), jnp.float32),
        pltpu.VMEM((_CHUNK + _LANES,), jnp.float32),
        pltpu.VMEM((_CHUNK,), jnp.float32),
        pltpu.VMEM((_CHUNK,), jnp.float32),
        pltpu.VMEM((_CHUNK,), jnp.float32),
        pltpu.VMEM((_CHUNK,), jnp.float32),
        pltpu.VMEM((_CHUNK,), jnp.float32),
        pltpu.VMEM((_CHUNK,), jnp.float32),
        pltpu.SemaphoreType.DMA,
        pltpu.SemaphoreType.DMA,
        pltpu.SemaphoreType.DMA,
        pltpu.SemaphoreType.DMA,
        pltpu.SemaphoreType.DMA,
    ],
    compiler_params=pltpu.CompilerParams(needs_layout_passes=False),
)


def kernel(x, y, z, phi_x):
    sub = phi_x[_SUB:, _SUB:, _SUB:, :]
    g0 = sub[..., 0].reshape(_CELLS)
    g1 = sub[..., 1].reshape(_CELLS)
    g2 = sub[..., 2].reshape(_CELLS)
    o0, o1, o2 = _sc_call(x, y, z, g0, g1, g2)
    return jnp.stack([o0, o1, o2], axis=-1)


# speedup vs baseline: 2.2589x; 1.0034x over previous
import jax
import jax.numpy as jnp
from jax import lax
from jax.experimental import pallas as pl
from jax.experimental.pallas import tpu as pltpu
from jax.experimental.pallas import tpu_sc as plsc

_N = 500000
_SUB = 30
_SG = 34
_CELLS = _SG * _SG * _SG

_NUM_CORES = 2
_NUM_SUBCORES = 16
_NW = _NUM_CORES * _NUM_SUBCORES
_LANES = 16

_CHUNK = 960
_NCHUNKS = 17
_PPT = _CHUNK * _NCHUNKS
_NPAD = _NW * _PPT
_GROUPS = _CHUNK // _LANES

_INV_SPACING = 30.5


def _weights(f):
    t = 1.0 - f
    s0 = t * t * t * (1.0 / 6.0)
    f2 = f * f
    f3 = f2 * f
    s1 = 0.5 * f3 - f2 + (2.0 / 3.0)
    s3 = f3 * (1.0 / 6.0)
    s2 = 1.0 - s0 - s1 - s3
    return s0, s1, s2, s3


def _sc_body(x_hbm, y_hbm, z_hbm, g0_hbm, g1_hbm, g2_hbm,
             o0_hbm, o1_hbm, o2_hbm,
             g0_v, g1_v, g2_v,
             x0_v, y0_v, z0_v, x1_v, y1_v, z1_v,
             t00_v, t01_v, t02_v, t10_v, t11_v, t12_v,
             sgrid, sin0, sin1, sout0, sout1):
    wid = lax.axis_index("s") * _NUM_CORES + lax.axis_index("c")
    base = wid * _PPT

    ins = [(x0_v, y0_v, z0_v, sin0), (x1_v, y1_v, z1_v, sin1)]
    outs = [(t00_v, t01_v, t02_v, sout0), (t10_v, t11_v, t12_v, sout1)]

    def issue_in(k):
        xv, yv, zv, sem = ins[k % 2]
        off = jnp.minimum(base + k * _CHUNK, _N - _CHUNK)
        return [
            pltpu.async_copy(x_hbm.at[pl.ds(off, _CHUNK)],
                             xv.at[pl.ds(0, _CHUNK)], sem),
            pltpu.async_copy(y_hbm.at[pl.ds(off, _CHUNK)],
                             yv.at[pl.ds(0, _CHUNK)], sem),
            pltpu.async_copy(z_hbm.at[pl.ds(off, _CHUNK)],
                             zv.at[pl.ds(0, _CHUNK)], sem),
        ]

    def issue_out(k):
        o0, o1, o2, sem = outs[k % 2]
        off = jnp.minimum(base + k * _CHUNK, _N - _CHUNK)
        return [
            pltpu.async_copy(o0, o0_hbm.at[pl.ds(off, _CHUNK)], sem),
            pltpu.async_copy(o1, o1_hbm.at[pl.ds(off, _CHUNK)], sem),
            pltpu.async_copy(o2, o2_hbm.at[pl.ds(off, _CHUNK)], sem),
        ]

    def compute_chunk(b):
        x_v, y_v, z_v, _ = ins[b]
        t0_v, t1_v, t2_v, _ = outs[b]

        def group_prep(s):
            xv = x_v[pl.ds(s, _LANES)]
            yv = y_v[pl.ds(s, _LANES)]
            zv = z_v[pl.ds(s, _LANES)]

            u = xv * _INV_SPACING + _INV_SPACING
            v = yv * _INV_SPACING + _INV_SPACING
            w = zv * _INV_SPACING + _INV_SPACING
            ix = u.astype(jnp.int32)
            iy = v.astype(jnp.int32)
            iz = w.astype(jnp.int32)
            fu = u - ix.astype(jnp.float32)
            fv = v - iy.astype(jnp.float32)
            fw = w - iz.astype(jnp.float32)

            su = _weights(fu)
            sv = _weights(fv)
            sw = _weights(fw)

            cell = (ix * _SG + iy) * _SG + iz - (
                (_SUB * _SG + _SUB) * _SG + _SUB)
            return su + sv + sw + (cell,)

        def group_body(g, carry):
            s = g * _LANES
            prep = carry
            nxt = group_prep(s + _LANES)
            su, sv, sw = prep[0:4], prep[4:8], prep[8:12]
            cell = prep[12]

            a0 = a1 = a2 = None
            for l in range(4):
                for m in range(4):
                    wlm = su[l] * sv[m]
                    row = cell + (l * _SG + m) * _SG
                    for n in range(4):
                        wt = wlm * sw[n]
                        idx = row + n
                        v0 = plsc.load_gather(g0_v, [idx])
                        v1 = plsc.load_gather(g1_v, [idx])
                        v2 = plsc.load_gather(g2_v, [idx])
                        if a0 is None:
                            a0 = wt * v0
                            a1 = wt * v1
                            a2 = wt * v2
                        else:
                            a0 = a0 + wt * v0
                            a1 = a1 + wt * v1
                            a2 = a2 + wt * v2

            t0_v[pl.ds(s, _LANES)] = a0
            t1_v[pl.ds(s, _LANES)] = a1
            t2_v[pl.ds(s, _LANES)] = a2
            return nxt

        lax.fori_loop(0, _GROUPS, group_body, group_prep(0))

    grid_copies = [
        pltpu.async_copy(g0_hbm, g0_v, sgrid),
        pltpu.async_copy(g1_hbm, g1_v, sgrid),
        pltpu.async_copy(g2_hbm, g2_v, sgrid),
    ]
    pending_in = issue_in(0)
    for h in grid_copies:
        h.wait()

    pending_out = {}
    for k in range(_NCHUNKS):
        for h in pending_in:
            h.wait()
        if k + 1 < _NCHUNKS:
            pending_in = issue_in(k + 1)
        if k - 2 in pending_out:
            for h in pending_out.pop(k - 2):
                h.wait()
        compute_chunk(k % 2)
        pending_out[k] = issue_out(k)
    for k in sorted(pending_out):
        for h in pending_out.pop(k):
            h.wait()


_sc_call = pl.kernel(
    _sc_body,
    out_type=(jax.ShapeDtypeStruct((_N,), jnp.float32),
              jax.ShapeDtypeStruct((_N,), jnp.float32),
              jax.ShapeDtypeStruct((_N,), jnp.float32)),
    mesh=plsc.VectorSubcoreMesh(
        core_axis_name="c", subcore_axis_name="s",
        num_cores=_NUM_CORES, num_subcores=_NUM_SUBCORES),
    scratch_types=[
        pltpu.VMEM((_CELLS,), jnp.float32),
        pltpu.VMEM((_CELLS,), jnp.float32),
        pltpu.VMEM((_CELLS,), jnp.float32),
        pltpu.VMEM((_CHUNK + _LANES,), jnp.float32),
        pltpu.VMEM((_CHUNK + _LANES,), jnp.float32),
        pltpu.VMEM((_CHUNK + _LANES,), jnp.float32),
        pltpu.VMEM((_CHUNK + _LANES,), jnp.float32),
        pltpu.VMEM((_CHUNK + _LANES,), jnp.float32),
        pltpu.VMEM((_CHUNK + _LANES,), jnp.float32),
        pltpu.VMEM((_CHUNK,), jnp.float32),
        pltpu.VMEM((_CHUNK,), jnp.float32),
        pltpu.VMEM((_CHUNK,), jnp.float32),
        pltpu.VMEM((_CHUNK,), jnp.float32),
        pltpu.VMEM((_CHUNK,), jnp.float32),
        pltpu.VMEM((_CHUNK,), jnp.float32),
        pltpu.SemaphoreType.DMA,
        pltpu.SemaphoreType.DMA,
        pltpu.SemaphoreType.DMA,
        pltpu.SemaphoreType.DMA,
        pltpu.SemaphoreType.DMA,
    ],
    compiler_params=pltpu.CompilerParams(
        needs_layout_passes=False,
        disable_bounds_checks=True,
        disable_semaphore_checks=True,
        skip_device_barrier=True,
    ),
)


def kernel(x, y, z, phi_x):
    sub = phi_x[_SUB:, _SUB:, _SUB:, :]
    g0 = sub[..., 0].reshape(_CELLS)
    g1 = sub[..., 1].reshape(_CELLS)
    g2 = sub[..., 2].reshape(_CELLS)
    o0, o1, o2 = _sc_call(x, y, z, g0, g1, g2)
    return jnp.stack([o0, o1, o2], axis=-1)


# speedup vs baseline: 2.3052x; 1.0205x over previous
import jax
import jax.numpy as jnp
from jax import lax
from jax.experimental import pallas as pl
from jax.experimental.pallas import tpu as pltpu
from jax.experimental.pallas import tpu_sc as plsc

_N = 500000
_SUB = 30
_SG = 34
_CELLS = _SG * _SG * _SG

_NUM_CORES = 2
_NUM_SUBCORES = 16
_NW = _NUM_CORES * _NUM_SUBCORES
_LANES = 16

_CHUNK = 960
_NCHUNKS = 17
_PPT = _CHUNK * _NCHUNKS
_NPAD = _NW * _PPT
_GROUPS = _CHUNK // _LANES

_INV_SPACING = 30.5


def _weights(f):
    t = 1.0 - f
    s0 = t * t * t * (1.0 / 6.0)
    f2 = f * f
    f3 = f2 * f
    s1 = 0.5 * f3 - f2 + (2.0 / 3.0)
    s3 = f3 * (1.0 / 6.0)
    s2 = 1.0 - s0 - s1 - s3
    return s0, s1, s2, s3


def _sc_body(x_hbm, y_hbm, z_hbm, g0_hbm, g1_hbm, g2_hbm,
             o0_hbm, o1_hbm, o2_hbm,
             g0_v, g1_v, g2_v,
             x0_v, y0_v, z0_v, x1_v, y1_v, z1_v,
             t00_v, t01_v, t02_v, t10_v, t11_v, t12_v,
             sgrid, sin0, sin1, sout0, sout1):
    wid = lax.axis_index("s") * _NUM_CORES + lax.axis_index("c")
    base = wid * _PPT

    ins = [(x0_v, y0_v, z0_v, sin0), (x1_v, y1_v, z1_v, sin1)]
    outs = [(t00_v, t01_v, t02_v, sout0), (t10_v, t11_v, t12_v, sout1)]

    def issue_in(k, b=None):
        xv, yv, zv, sem = ins[b if b is not None else k % 2]
        off = jnp.minimum(base + k * _CHUNK, _N - _CHUNK)
        return [
            pltpu.async_copy(x_hbm.at[pl.ds(off, _CHUNK)],
                             xv.at[pl.ds(0, _CHUNK)], sem),
            pltpu.async_copy(y_hbm.at[pl.ds(off, _CHUNK)],
                             yv.at[pl.ds(0, _CHUNK)], sem),
            pltpu.async_copy(z_hbm.at[pl.ds(off, _CHUNK)],
                             zv.at[pl.ds(0, _CHUNK)], sem),
        ]

    def issue_out(k, b=None):
        o0, o1, o2, sem = outs[b if b is not None else k % 2]
        off = jnp.minimum(base + k * _CHUNK, _N - _CHUNK)
        return [
            pltpu.async_copy(o0, o0_hbm.at[pl.ds(off, _CHUNK)], sem),
            pltpu.async_copy(o1, o1_hbm.at[pl.ds(off, _CHUNK)], sem),
            pltpu.async_copy(o2, o2_hbm.at[pl.ds(off, _CHUNK)], sem),
        ]

    def compute_chunk(b):
        x_v, y_v, z_v, _ = ins[b]
        t0_v, t1_v, t2_v, _ = outs[b]

        def group_prep(s):
            xv = x_v[pl.ds(s, _LANES)]
            yv = y_v[pl.ds(s, _LANES)]
            zv = z_v[pl.ds(s, _LANES)]

            u = xv * _INV_SPACING + _INV_SPACING
            v = yv * _INV_SPACING + _INV_SPACING
            w = zv * _INV_SPACING + _INV_SPACING
            ix = u.astype(jnp.int32)
            iy = v.astype(jnp.int32)
            iz = w.astype(jnp.int32)
            fu = u - ix.astype(jnp.float32)
            fv = v - iy.astype(jnp.float32)
            fw = w - iz.astype(jnp.float32)

            su = _weights(fu)
            sv = _weights(fv)
            sw = _weights(fw)

            cell = (ix * _SG + iy) * _SG + iz - (
                (_SUB * _SG + _SUB) * _SG + _SUB)
            return su + sv + sw + (cell,)

        def group_body(g, carry):
            s = g * _LANES
            prep = carry
            nxt = group_prep(s + _LANES)
            su, sv, sw = prep[0:4], prep[4:8], prep[8:12]
            cell = prep[12]

            a0 = a1 = a2 = None
            for l in range(4):
                for m in range(4):
                    wlm = su[l] * sv[m]
                    row = cell + (l * _SG + m) * _SG
                    for n in range(4):
                        wt = wlm * sw[n]
                        idx = row + n
                        v0 = plsc.load_gather(g0_v, [idx])
                        v1 = plsc.load_gather(g1_v, [idx])
                        v2 = plsc.load_gather(g2_v, [idx])
                        if a0 is None:
                            a0 = wt * v0
                            a1 = wt * v1
                            a2 = wt * v2
                        else:
                            a0 = a0 + wt * v0
                            a1 = a1 + wt * v1
                            a2 = a2 + wt * v2

            t0_v[pl.ds(s, _LANES)] = a0
            t1_v[pl.ds(s, _LANES)] = a1
            t2_v[pl.ds(s, _LANES)] = a2
            return nxt

        lax.fori_loop(0, _GROUPS, group_body, group_prep(0))

    grid_copies = [
        pltpu.async_copy(g0_hbm, g0_v, sgrid),
        pltpu.async_copy(g1_hbm, g1_v, sgrid),
        pltpu.async_copy(g2_hbm, g2_v, sgrid),
    ]
    pending_in = issue_in(0)
    for h in grid_copies:
        h.wait()

    def wait_in(b):
        xv, yv, zv, sem = ins[b]
        for hbm, ref in ((x_hbm, xv), (y_hbm, yv), (z_hbm, zv)):
            pltpu.make_async_copy(hbm.at[pl.ds(0, _CHUNK)],
                                  ref.at[pl.ds(0, _CHUNK)], sem).wait()

    def wait_out(b):
        o0, o1, o2, sem = outs[b]
        for src_v, hbm in ((o0, o0_hbm), (o1, o1_hbm), (o2, o2_hbm)):
            pltpu.make_async_copy(src_v, hbm.at[pl.ds(0, _CHUNK)],
                                  sem).wait()

    for k in (0, 1):
        wait_in(k % 2)
        issue_in(k + 1)
        compute_chunk(k % 2)
        issue_out(k)

    def pair_body(i, carry):
        k0 = 2 * i
        for d in (0, 1):
            k = k0 + d
            b = d
            wait_in(b)
            issue_in(k + 1, 1 - b)
            wait_out(b)
            compute_chunk(b)
            issue_out(k, b)
        return carry

    lax.fori_loop(1, 8, pair_body, 0)

    wait_in(0)
    wait_out(0)
    compute_chunk(0)
    issue_out(16)
    wait_out(1)
    wait_out(0)


_sc_call = pl.kernel(
    _sc_body,
    out_type=(jax.ShapeDtypeStruct((_N,), jnp.float32),
              jax.ShapeDtypeStruct((_N,), jnp.float32),
              jax.ShapeDtypeStruct((_N,), jnp.float32)),
    mesh=plsc.VectorSubcoreMesh(
        core_axis_name="c", subcore_axis_name="s",
        num_cores=_NUM_CORES, num_subcores=_NUM_SUBCORES),
    scratch_types=[
        pltpu.VMEM((_CELLS,), jnp.float32),
        pltpu.VMEM((_CELLS,), jnp.float32),
        pltpu.VMEM((_CELLS,), jnp.float32),
        pltpu.VMEM((_CHUNK + _LANES,), jnp.float32),
        pltpu.VMEM((_CHUNK + _LANES,), jnp.float32),
        pltpu.VMEM((_CHUNK + _LANES,), jnp.float32),
        pltpu.VMEM((_CHUNK + _LANES,), jnp.float32),
        pltpu.VMEM((_CHUNK + _LANES,), jnp.float32),
        pltpu.VMEM((_CHUNK + _LANES,), jnp.float32),
        pltpu.VMEM((_CHUNK,), jnp.float32),
        pltpu.VMEM((_CHUNK,), jnp.float32),
        pltpu.VMEM((_CHUNK,), jnp.float32),
        pltpu.VMEM((_CHUNK,), jnp.float32),
        pltpu.VMEM((_CHUNK,), jnp.float32),
        pltpu.VMEM((_CHUNK,), jnp.float32),
        pltpu.SemaphoreType.DMA,
        pltpu.SemaphoreType.DMA,
        pltpu.SemaphoreType.DMA,
        pltpu.SemaphoreType.DMA,
        pltpu.SemaphoreType.DMA,
    ],
    compiler_params=pltpu.CompilerParams(needs_layout_passes=False),
)


def kernel(x, y, z, phi_x):
    sub = phi_x[_SUB:, _SUB:, _SUB:, :]
    g0 = sub[..., 0].reshape(_CELLS)
    g1 = sub[..., 1].reshape(_CELLS)
    g2 = sub[..., 2].reshape(_CELLS)
    o0, o1, o2 = _sc_call(x, y, z, g0, g1, g2)
    return jnp.stack([o0, o1, o2], axis=-1)
